# Initial kernel scaffold; baseline (speedup 1.0000x reference)
import jax
import jax.numpy as jnp
from jax.experimental import pallas as pl


def kernel(node_features, edge_index, batch, static_features, W_proj, b_proj, ln1_g, ln1_b, W1, att_src1, att_dst1, b1, W2, att_src2, att_dst2, b2, Wc1, bc1, lnc_g, lnc_b, Wc2, bc2):
    raise NotImplementedError("write your pallas kernel here")


# speedup vs baseline: 5.6726x; 5.6726x over previous
import functools
import jax
import jax.numpy as jnp
from jax import lax
from jax.experimental import pallas as pl
from jax.experimental.pallas import tpu as pltpu
from jax.experimental.pallas import tpu_sc as plsc

N = 10000
E = 160000
B = 16
IN_DIM = 768
HID = 256
HEADS = 4
NCLS = 5
STATIC = 12

NPAD = 10240
NB = NPAD // 256
EPAD = 172032
EB = 128
E8 = EPAD // 8
E16 = EPAD // 16
E32 = EPAD // 32
ROWS_T = NPAD // 16
NCH1 = 16
NCH2 = 4


@functools.cache
def _mesh():
    return plsc.VectorSubcoreMesh(
        core_axis_name="c", subcore_axis_name="s", num_cores=2,
        num_subcores=16)


def _elu(x):
    return jnp.where(x > 0, x, jnp.exp(jnp.minimum(x, 0.0)) - 1.0)


def _proj_body(nf, wp, bp, g1, b1ln, w1, asm, adm, *outs):
    h1c = outs[0:NCH1]
    as1, ad1 = outs[NCH1], outs[NCH1 + 1]
    i = pl.program_id(0)
    x = jnp.dot(nf[...], wp[...], preferred_element_type=jnp.float32) + bp[...]
    mu = jnp.mean(x, axis=1, keepdims=True)
    var = jnp.mean((x - mu) * (x - mu), axis=1, keepdims=True)
    x = (x - mu) / jnp.sqrt(var + 1e-5) * g1[...] + b1ln[...]
    x = jnp.maximum(x, 0.0)
    rows = i * 256 + lax.broadcasted_iota(jnp.int32, (256, 1), 0)
    x = jnp.where(rows < N, x, 0.0)
    h1 = jnp.dot(x, w1[...], preferred_element_type=jnp.float32)
    for c in range(NCH1):
        h1c[c][...] = h1[:, c * 64:(c + 1) * 64]
    as1[...] = jnp.dot(h1, asm[...], preferred_element_type=jnp.float32)
    ad1[...] = jnp.dot(h1, adm[...], preferred_element_type=jnp.float32)


def _run_proj(nfp, w_proj, b_proj, ln1_g, ln1_b, w1, asm, adm):
    full = lambda shape: pl.BlockSpec(shape, lambda i: (0,) * len(shape))
    outs = pl.pallas_call(
        _proj_body,
        grid=(NB,),
        in_specs=[
            pl.BlockSpec((256, IN_DIM), lambda i: (i, 0)),
            full((IN_DIM, HID)), full((1, HID)), full((1, HID)),
            full((1, HID)), full((HID, 4 * HID)),
            full((4 * HID, 4)), full((4 * HID, 4)),
        ],
        out_specs=[pl.BlockSpec((256, 64), lambda i: (i, 0))] * NCH1
        + [pl.BlockSpec((256, 4), lambda i: (i, 0))] * 2,
        out_shape=[jax.ShapeDtypeStruct((NPAD, 64), jnp.float32)] * NCH1
        + [jax.ShapeDtypeStruct((NPAD, 4), jnp.float32)] * 2,
    )(nfp, w_proj, b_proj.reshape(1, HID), ln1_g.reshape(1, HID),
      ln1_b.reshape(1, HID), w1, asm, adm)
    return outs[0:NCH1], outs[NCH1], outs[NCH1 + 1]


def _w_block(ash, adh, srcb, dstb2, wblk, rbuf, hcol):
    lane = lax.broadcasted_iota(jnp.int32, (16,), 0)
    hvec = jnp.full((16,), hcol, jnp.int32)

    def body(i, _):
        sl = pl.ds(i * 16, 16)
        s = srcb[sl]
        d = dstb2[0, sl]
        av = plsc.load_gather(ash, [s]) + plsc.load_gather(adh, [d])
        av = jnp.where(av > 0, av, 0.2 * av)
        w = jnp.exp(av)
        wblk[sl] = w
        plsc.store_scatter(rbuf, [i * 16 + lane, hvec], w)
        return 0
    lax.fori_loop(0, EB // 16, body, 0)


def _f_block(h1ref, w1f, srcp, dstp2, sidx, didx2, wbuf, feat, acc, sem,
             ebase, erow, woff):
    pltpu.sync_copy(srcp.at[pl.ds(ebase, EB)], sidx)
    pltpu.sync_copy(dstp2.at[pl.ds(erow, 1)], didx2)
    pltpu.sync_copy(w1f.at[pl.ds(woff, EB)], wbuf)
    pltpu.async_copy(h1ref.at[sidx], feat, sem).wait()

    def mul(i, _):
        wv16 = wbuf[pl.ds(i * 16, 16)]
        for j in range(16):
            k = i * 16 + j
            wv = jnp.full((16,), wv16[j], jnp.float32)
            for m in range(4):
                sl = pl.ds(m * 16, 16)
                feat[k, sl] = feat[k, sl] * wv
        return 0
    lax.fori_loop(0, EB // 16, mul, 0)
    pltpu.sync_copy(feat, acc.at[didx2.at[0]], add=True)


def _gat1_body(as1t, ad1t, srcp, dstp2, zh64, zh16, *rest):
    h1c = rest[0:NCH1]
    w1f, den1 = rest[NCH1], rest[NCH1 + 1]
    out1 = rest[NCH1 + 2:NCH1 + 2 + NCH1]
    (ash, adh, srcb, dstb2, wblk, rbuf, sidx, didx2, wbuf, feat,
     acc, dacc, sem) = rest[NCH1 + 2 + NCH1:]
    cid = lax.axis_index("c")
    sid = lax.axis_index("s")
    row0 = sid * ROWS_T

    pltpu.sync_copy(zh16.at[pl.ds(0, EB)], rbuf)
    pltpu.sync_copy(zh16, dacc.at[pl.ds(row0, ROWS_T)])
    plsc.subcore_barrier()

    for hs in range(4):
        @pl.when(cid * 2 + sid // 8 == hs)
        def _():
            pltpu.sync_copy(as1t.at[hs], ash)
            pltpu.sync_copy(ad1t.at[hs], adh)
    r = sid % 8
    g = sid // 8
    hq = cid * 2 + g
    base = r * E8

    def wblock(b, _):
        e0 = base + b * EB
        pltpu.sync_copy(srcp.at[pl.ds(e0, EB)], srcb)
        pltpu.sync_copy(dstp2.at[pl.ds(r * (E8 // EB) + b, 1)], dstb2)
        _w_block(ash, adh, srcb, dstb2, wblk, rbuf, hq)
        pltpu.sync_copy(wblk, w1f.at[pl.ds(hq * EPAD + e0, EB)])
        pltpu.sync_copy(rbuf, dacc.at[dstb2.at[0]], add=True)
        return 0
    lax.fori_loop(0, E8 // EB, wblock, 0)
    plsc.subcore_barrier()

    pltpu.sync_copy(dacc.at[pl.ds(row0, ROWS_T)],
                    den1.at[pl.ds(cid * NPAD + row0, ROWS_T)])

    for step in range(8):
        pltpu.sync_copy(zh64, acc.at[pl.ds(row0, ROWS_T)])
        plsc.subcore_barrier()
        for cc in range(2):
            c = cc * 8 + step
            h = c // 4

            @pl.when(cid == cc)
            def _(c=c, h=h):
                def fblock(b, _):
                    ebase = sid * E16 + b * EB
                    _f_block(h1c[c], w1f, srcp, dstp2, sidx, didx2, wbuf,
                             feat, acc, sem, ebase, sid * (E16 // EB) + b,
                             h * EPAD + ebase)
                    return 0
                lax.fori_loop(0, E16 // EB, fblock, 0)
        plsc.subcore_barrier()
        for cc in range(2):
            c = cc * 8 + step

            @pl.when(cid == cc)
            def _(c=c):
                pltpu.sync_copy(acc.at[pl.ds(row0, ROWS_T)],
                                out1[c].at[pl.ds(row0, ROWS_T)])
        plsc.subcore_barrier()


def _run_gat1(as1t, ad1t, srcp, dstp2, zh64, zh16, h1c):
    kern = pl.kernel(
        _gat1_body,
        out_type=[jax.ShapeDtypeStruct((4 * EPAD,), jnp.float32),
                  jax.ShapeDtypeStruct((2 * NPAD, 16), jnp.float32)]
        + [jax.ShapeDtypeStruct((NPAD, 64), jnp.float32)] * NCH1,
        mesh=_mesh(),
        compiler_params=pltpu.CompilerParams(needs_layout_passes=False, use_tc_tiling_on_sc=False),
        scratch_types=[
            pltpu.VMEM((NPAD,), jnp.float32),
            pltpu.VMEM((NPAD,), jnp.float32),
            pltpu.VMEM((EB,), jnp.int32),
            pltpu.VMEM((1, EB), jnp.int32),
            pltpu.VMEM((EB,), jnp.float32),
            pltpu.VMEM((EB, 16), jnp.float32),
            pltpu.VMEM((EB,), jnp.int32),
            pltpu.VMEM((1, EB), jnp.int32),
            pltpu.VMEM((EB,), jnp.float32),
            pltpu.VMEM((EB, 64), jnp.float32),
            pltpu.VMEM_SHARED((NPAD, 64), jnp.float32),
            pltpu.VMEM_SHARED((NPAD, 16), jnp.float32),
            pltpu.SemaphoreType.DMA,
        ],
    )
    return kern(as1t, ad1t, srcp, dstp2, zh64, zh16, *h1c)


def _mid_body(*refs):
    o1 = refs[0:NCH1]
    dnb, b1r, w2, a2s, a2d = refs[NCH1:NCH1 + 5]
    outs = refs[NCH1 + 5:]
    h2c = outs[0:NCH2]
    as2, ad2 = outs[NCH2], outs[NCH2 + 1]
    i = pl.program_id(0)
    cols = []
    for c in range(NCH1):
        dnh = dnb[c // 4, :][:, None]
        xc = o1[c][...] / (dnh + 1e-16) + b1r[0, c * 64:(c + 1) * 64][None, :]
        cols.append(_elu(xc))
    x2 = jnp.concatenate(cols, axis=1)
    rows = i * 256 + lax.broadcasted_iota(jnp.int32, (256, 1), 0)
    x2 = jnp.where(rows < N, x2, 0.0)
    h2 = jnp.dot(x2, w2[...], preferred_element_type=jnp.float32)
    for c in range(NCH2):
        h2c[c][...] = h2[:, c * 64:(c + 1) * 64]
    as2[...] = jnp.dot(h2, a2s[...], preferred_element_type=jnp.float32)
    ad2[...] = jnp.dot(h2, a2d[...], preferred_element_type=jnp.float32)


def _run_mid(out1, den1s, b1, w2, att_src2, att_dst2):
    full = lambda shape: pl.BlockSpec(shape, lambda i: (0,) * len(shape))
    outs = pl.pallas_call(
        _mid_body,
        grid=(NB,),
        in_specs=[pl.BlockSpec((256, 64), lambda i: (i, 0))] * NCH1
        + [pl.BlockSpec((4, 256), lambda i: (0, i)),
           full((1, 4 * HID)), full((4 * HID, HID)),
           full((HID, 1)), full((HID, 1))],
        out_specs=[pl.BlockSpec((256, 64), lambda i: (i, 0))] * NCH2
        + [pl.BlockSpec((256, 1), lambda i: (i, 0))] * 2,
        out_shape=[jax.ShapeDtypeStruct((NPAD, 64), jnp.float32)] * NCH2
        + [jax.ShapeDtypeStruct((NPAD, 1), jnp.float32)] * 2,
    )(*out1, den1s, b1.reshape(1, 4 * HID), w2,
      att_src2.reshape(HID, 1), att_dst2.reshape(HID, 1))
    return outs[0:NCH2], outs[NCH2], outs[NCH2 + 1]


def _gat2_body(as2f, ad2f, srcp, dstp2, zh64, zh16, h2c0, h2c1, h2c2, h2c3,
               w2f, den2, o2a, o2b, o2c, o2d,
               ash, adh, srcb, dstb2, wblk, rbuf, sidx, didx2, wbuf,
               feat, acc, dacc, sem):
    h2c = (h2c0, h2c1, h2c2, h2c3)
    out2 = (o2a, o2b, o2c, o2d)
    cid = lax.axis_index("c")
    sid = lax.axis_index("s")
    row0 = sid * ROWS_T

    pltpu.sync_copy(zh16.at[pl.ds(0, EB)], rbuf)
    pltpu.sync_copy(zh16, dacc.at[pl.ds(row0, ROWS_T)])
    plsc.subcore_barrier()

    pltpu.sync_copy(as2f, ash)
    pltpu.sync_copy(ad2f, adh)
    r32 = cid * 16 + sid
    base = r32 * E32

    def wblock(b, _):
        e0 = base + b * EB
        pltpu.sync_copy(srcp.at[pl.ds(e0, EB)], srcb)
        pltpu.sync_copy(dstp2.at[pl.ds(r32 * (E32 // EB) + b, 1)], dstb2)
        _w_block(ash, adh, srcb, dstb2, wblk, rbuf, 0)
        pltpu.sync_copy(wblk, w2f.at[pl.ds(e0, EB)])
        pltpu.sync_copy(rbuf, dacc.at[dstb2.at[0]], add=True)
        return 0
    lax.fori_loop(0, E32 // EB, wblock, 0)
    plsc.subcore_barrier()

    pltpu.sync_copy(dacc.at[pl.ds(row0, ROWS_T)],
                    den2.at[pl.ds(cid * NPAD + row0, ROWS_T)])

    for step in range(2):
        pltpu.sync_copy(zh64, acc.at[pl.ds(row0, ROWS_T)])
        plsc.subcore_barrier()
        for cc in range(2):
            c = cc * 2 + step

            @pl.when(cid == cc)
            def _(c=c):
                def fblock(b, _):
                    ebase = sid * E16 + b * EB
                    _f_block(h2c[c], w2f, srcp, dstp2, sidx, didx2, wbuf,
                             feat, acc, sem, ebase, sid * (E16 // EB) + b,
                             ebase)
                    return 0
                lax.fori_loop(0, E16 // EB, fblock, 0)
        plsc.subcore_barrier()
        for cc in range(2):
            c = cc * 2 + step

            @pl.when(cid == cc)
            def _(c=c):
                pltpu.sync_copy(acc.at[pl.ds(row0, ROWS_T)],
                                out2[c].at[pl.ds(row0, ROWS_T)])
        plsc.subcore_barrier()


def _run_gat2(as2f, ad2f, srcp, dstp2, zh64, zh16, h2c):
    kern = pl.kernel(
        _gat2_body,
        out_type=[jax.ShapeDtypeStruct((EPAD,), jnp.float32),
                  jax.ShapeDtypeStruct((2 * NPAD, 16), jnp.float32)]
        + [jax.ShapeDtypeStruct((NPAD, 64), jnp.float32)] * NCH2,
        mesh=_mesh(),
        compiler_params=pltpu.CompilerParams(needs_layout_passes=False, use_tc_tiling_on_sc=False),
        scratch_types=[
            pltpu.VMEM((NPAD,), jnp.float32),
            pltpu.VMEM((NPAD,), jnp.float32),
            pltpu.VMEM((EB,), jnp.int32),
            pltpu.VMEM((1, EB), jnp.int32),
            pltpu.VMEM((EB,), jnp.float32),
            pltpu.VMEM((EB, 16), jnp.float32),
            pltpu.VMEM((EB,), jnp.int32),
            pltpu.VMEM((1, EB), jnp.int32),
            pltpu.VMEM((EB,), jnp.float32),
            pltpu.VMEM((EB, 64), jnp.float32),
            pltpu.VMEM_SHARED((NPAD, 64), jnp.float32),
            pltpu.VMEM_SHARED((NPAD, 16), jnp.float32),
            pltpu.SemaphoreType.DMA,
        ],
    )
    return kern(as2f, ad2f, srcp, dstp2, zh64, zh16, *h2c)


def _head_body(*refs):
    o2 = refs[0:NCH2]
    dnb, b2r, bat, stf, wc1, bc1, lg, lb, wc2, bc2 = refs[NCH2:NCH2 + 10]
    out, pooled, cnt = refs[NCH2 + 10:]
    i = pl.program_id(0)

    @pl.when(i == 0)
    def _():
        pooled[...] = jnp.zeros((B, HID), jnp.float32)
        cnt[...] = jnp.zeros((B, 128), jnp.float32)

    d2 = (dnb[0, :] + dnb[1, :])[:, None]
    x3 = jnp.concatenate([o2[c][...] for c in range(NCH2)], axis=1)
    x3 = _elu(x3 / (d2 + 1e-16) + b2r[...])
    onehot = (bat[0, 0, :][None, :] ==
              lax.broadcasted_iota(jnp.int32, (B, 256), 0)).astype(jnp.float32)
    pooled[...] += jnp.dot(onehot, x3, preferred_element_type=jnp.float32)
    s = jnp.sum(onehot, axis=1, keepdims=True)
    cnt[...] += jnp.broadcast_to(s, (B, 128))

    @pl.when(i == NB - 1)
    def _():
        counts = cnt[:, 0:1]
        pm = pooled[...] / jnp.maximum(counts, 1.0)
        z = (jnp.dot(pm, wc1[0:HID, :], preferred_element_type=jnp.float32)
             + jnp.dot(stf[...], wc1[HID:HID + STATIC, :],
                       preferred_element_type=jnp.float32) + bc1[...])
        mu = jnp.mean(z, axis=1, keepdims=True)
        var = jnp.mean((z - mu) * (z - mu), axis=1, keepdims=True)
        z = (z - mu) / jnp.sqrt(var + 1e-5) * lg[...] + lb[...]
        z = jnp.maximum(z, 0.0)
        out[...] = jnp.dot(z, wc2[...],
                           preferred_element_type=jnp.float32) + bc2[...]


def _run_head(out2, den2s, b2, bat3, stf, wc1, bc1, lg, lb, wc2, bc2):
    full = lambda shape: pl.BlockSpec(shape, lambda i: (0,) * len(shape))
    return pl.pallas_call(
        _head_body,
        grid=(NB,),
        in_specs=[pl.BlockSpec((256, 64), lambda i: (i, 0))] * NCH2
        + [pl.BlockSpec((2, 256), lambda i: (0, i)),
           full((1, HID)),
           pl.BlockSpec((1, 1, 256), lambda i: (i, 0, 0)),
           full((B, STATIC)), full((HID + STATIC, 128)), full((1, 128)),
           full((1, 128)), full((1, 128)), full((128, NCLS)),
           full((1, NCLS))],
        out_specs=pl.BlockSpec((B, NCLS), lambda i: (0, 0)),
        out_shape=jax.ShapeDtypeStruct((B, NCLS), jnp.float32),
        scratch_shapes=[pltpu.VMEM((B, HID), jnp.float32),
                        pltpu.VMEM((B, 128), jnp.float32)],
    )(*out2, den2s, b2.reshape(1, HID), bat3, stf, wc1,
      bc1.reshape(1, 128), lg.reshape(1, 128), lb.reshape(1, 128),
      wc2, bc2.reshape(1, NCLS))


def kernel(node_features, edge_index, batch, static_features, W_proj, b_proj,
           ln1_g, ln1_b, W1, att_src1, att_dst1, b1, W2, att_src2, att_dst2,
           b2, Wc1, bc1, lnc_g, lnc_b, Wc2, bc2):
    f32 = jnp.float32
    nfp = jnp.pad(node_features, ((0, NPAD - N), (0, 0)))
    loop = jnp.arange(N, dtype=jnp.int32)
    npadidx = jnp.full((EPAD - E - N,), NPAD - 1, jnp.int32)
    srcp = jnp.concatenate([edge_index[0].astype(jnp.int32), loop, npadidx])
    dstp = jnp.concatenate([edge_index[1].astype(jnp.int32), loop, npadidx])
    dstp2 = dstp.reshape(EPAD // EB, EB)
    batp = jnp.concatenate([batch.astype(jnp.int32),
                            jnp.full((NPAD - N,), B, jnp.int32)])
    bat3 = batp.reshape(NB, 1, 256)
    zh64 = jnp.zeros((ROWS_T, 64), f32)
    zh16 = jnp.zeros((ROWS_T, 16), f32)

    eye4 = jnp.eye(HEADS, dtype=f32)
    asm = (att_src1[:, :, None] * eye4[:, None, :]).reshape(HEADS * HID, HEADS)
    adm = (att_dst1[:, :, None] * eye4[:, None, :]).reshape(HEADS * HID, HEADS)

    h1c, as1, ad1 = _run_proj(nfp, W_proj, b_proj, ln1_g, ln1_b, W1, asm, adm)
    as1t = as1.T.reshape(HEADS, NPAD)
    ad1t = ad1.T.reshape(HEADS, NPAD)

    g1 = _run_gat1(as1t, ad1t, srcp, dstp2, zh64, zh16, h1c)
    den1f, out1 = g1[1], g1[2:2 + NCH1]
    den1r = den1f.reshape(2, NPAD, 16)
    den1s = jnp.stack([den1r[0, :, 0], den1r[0, :, 1],
                       den1r[1, :, 2], den1r[1, :, 3]])

    h2c, as2, ad2 = _run_mid(out1, den1s, b1, W2, att_src2, att_dst2)
    as2f = as2[:, 0]
    ad2f = ad2[:, 0]

    g2 = _run_gat2(as2f, ad2f, srcp, dstp2, zh64, zh16, h2c)
    den2f, out2 = g2[1], g2[2:2 + NCH2]
    den2s = den2f[:, 0].reshape(2, NPAD)

    return _run_head(out2, den2s, b2, bat3, static_features, Wc1, bc1,
                     lnc_g, lnc_b, Wc2, bc2)


# speedup vs baseline: 11.6220x; 2.0488x over previous
import functools
import jax
import jax.numpy as jnp
from jax import lax
from jax.experimental import pallas as pl
from jax.experimental.pallas import tpu as pltpu
from jax.experimental.pallas import tpu_sc as plsc

N = 10000
E = 160000
B = 16
IN_DIM = 768
HID = 256
HEADS = 4
NCLS = 5
STATIC = 12

NPAD = 10240
NB = NPAD // 256
EPAD = 172032
EB = 128
E8 = EPAD // 8
E16 = EPAD // 16
E32 = EPAD // 32
ROWS_T = NPAD // 16
NCH1 = 16
NCH2 = 4


@functools.cache
def _mesh():
    return plsc.VectorSubcoreMesh(
        core_axis_name="c", subcore_axis_name="s", num_cores=2,
        num_subcores=16)


def _elu(x):
    return jnp.where(x > 0, x, jnp.exp(jnp.minimum(x, 0.0)) - 1.0)


def _proj_body(nf, wp, bp, g1, b1ln, w1, asm, adm, *outs):
    h1c = outs[0:NCH1]
    as1, ad1 = outs[NCH1], outs[NCH1 + 1]
    i = pl.program_id(0)
    x = jnp.dot(nf[...], wp[...], preferred_element_type=jnp.float32) + bp[...]
    mu = jnp.mean(x, axis=1, keepdims=True)
    var = jnp.mean((x - mu) * (x - mu), axis=1, keepdims=True)
    x = (x - mu) / jnp.sqrt(var + 1e-5) * g1[...] + b1ln[...]
    x = jnp.maximum(x, 0.0)
    rows = i * 256 + lax.broadcasted_iota(jnp.int32, (256, 1), 0)
    x = jnp.where(rows < N, x, 0.0)
    h1 = jnp.dot(x, w1[...], preferred_element_type=jnp.float32)
    for c in range(NCH1):
        h1c[c][...] = h1[:, c * 64:(c + 1) * 64].astype(jnp.bfloat16)
    as1[...] = jnp.dot(h1, asm[...], preferred_element_type=jnp.float32)
    ad1[...] = jnp.dot(h1, adm[...], preferred_element_type=jnp.float32)


def _run_proj(nfp, w_proj, b_proj, ln1_g, ln1_b, w1, asm, adm):
    full = lambda shape: pl.BlockSpec(shape, lambda i: (0,) * len(shape))
    outs = pl.pallas_call(
        _proj_body,
        grid=(NB,),
        in_specs=[
            pl.BlockSpec((256, IN_DIM), lambda i: (i, 0)),
            full((IN_DIM, HID)), full((1, HID)), full((1, HID)),
            full((1, HID)), full((HID, 4 * HID)),
            full((4 * HID, 4)), full((4 * HID, 4)),
        ],
        out_specs=[pl.BlockSpec((256, 64), lambda i: (i, 0))] * NCH1
        + [pl.BlockSpec((256, 4), lambda i: (i, 0))] * 2,
        out_shape=[jax.ShapeDtypeStruct((NPAD, 64), jnp.bfloat16)] * NCH1
        + [jax.ShapeDtypeStruct((NPAD, 4), jnp.float32)] * 2,
    )(nfp, w_proj, b_proj.reshape(1, HID), ln1_g.reshape(1, HID),
      ln1_b.reshape(1, HID), w1, asm, adm)
    return outs[0:NCH1], outs[NCH1], outs[NCH1 + 1]


def _w_inner(ash, adh, sidxf, didxf, wbuff, rbuf, hcol, b):
    lane = lax.broadcasted_iota(jnp.int32, (16,), 0)
    hvec = jnp.full((16,), hcol, jnp.int32)

    def body(i, _):
        sl = pl.ds(b * EB + i * 16, 16)
        s = sidxf[sl]
        d = didxf[b, pl.ds(i * 16, 16)]
        av = plsc.load_gather(ash, [s]) + plsc.load_gather(adh, [d])
        av = jnp.where(av > 0, av, 0.2 * av)
        w = jnp.exp(av)
        wbuff[sl] = w
        plsc.store_scatter(rbuf, [i * 16 + lane, hvec], w)
        return 0
    lax.fori_loop(0, EB // 16, body, 0)


def _w_range(ash, adh, srcp, dstp2, sidxf, didxf, wbuff, rbuf0, rbuf1,
             wsem0, wsem1, dacc, hcol, e0, nblk):
    ne = nblk * EB
    pltpu.sync_copy(srcp.at[pl.ds(e0, ne)], sidxf.at[pl.ds(0, ne)])
    pltpu.sync_copy(dstp2.at[pl.ds(e0 // EB, nblk)], didxf.at[pl.ds(0, nblk)])

    def pair(p, _):
        b0 = 2 * p
        b1 = b0 + 1

        @pl.when(p > 0)
        def _():
            pltpu.make_async_copy(rbuf0, dacc.at[didxf.at[0]], wsem0).wait()
        _w_inner(ash, adh, sidxf, didxf, wbuff, rbuf0, hcol, b0)
        pltpu.async_copy(rbuf0, dacc.at[didxf.at[b0]], wsem0, add=True)

        @pl.when(p > 0)
        def _():
            pltpu.make_async_copy(rbuf1, dacc.at[didxf.at[0]], wsem1).wait()
        _w_inner(ash, adh, sidxf, didxf, wbuff, rbuf1, hcol, b1)
        pltpu.async_copy(rbuf1, dacc.at[didxf.at[b1]], wsem1, add=True)
        return 0
    lax.fori_loop(0, nblk // 2, pair, 0)
    pltpu.make_async_copy(rbuf0, dacc.at[didxf.at[0]], wsem0).wait()
    pltpu.make_async_copy(rbuf1, dacc.at[didxf.at[0]], wsem1).wait()


def _mul_block(featb, fout, wbuff, boff):
    def mul(i, _):
        wv16 = wbuff[pl.ds(boff + i * 16, 16)]
        for j in range(16):
            k = i * 16 + j
            wv = jnp.full((16,), wv16[j], jnp.float32)
            for m in range(2):
                pk = featb[k, pl.ds(m * 32, 32)]
                a, b = plsc.unpack(pk, format=plsc.PackFormat.INTERLEAVED)
                fout[k, pl.ds(m * 32, 16)] = a * wv
                fout[k, pl.ds(m * 32 + 16, 16)] = b * wv
        return 0
    lax.fori_loop(0, EB // 16, mul, 0)


def _f_step(h1ref, wf, srcp, dstp2, sidxf, didxf, wbuff, featb0, featb1,
            fout0, fout1, gsem0, gsem1, ssem0, ssem1, acc, sid, woff_base):
    def gsrc(b):
        return h1ref.at[sidxf.at[pl.ds(b * EB, EB)]]

    def half(hh, _):
        ebase0 = sid * E16 + hh * E32
        pltpu.sync_copy(srcp.at[pl.ds(ebase0, E32)], sidxf)
        pltpu.sync_copy(wf.at[pl.ds(woff_base + ebase0, E32)], wbuff)
        pltpu.sync_copy(dstp2.at[pl.ds(sid * (E16 // EB) + hh * (E32 // EB),
                                       E32 // EB)], didxf)

        def pair(p, _):
            b0 = 2 * p
            b1 = b0 + 1
            pltpu.async_copy(gsrc(b0), featb0, gsem0)
            pltpu.async_copy(gsrc(b1), featb1, gsem1)

            @pl.when(p > 0)
            def _():
                pltpu.make_async_copy(fout0, acc.at[didxf.at[0]],
                                      ssem0).wait()
            pltpu.make_async_copy(gsrc(b0), featb0, gsem0).wait()
            _mul_block(featb0, fout0, wbuff, b0 * EB)
            pltpu.async_copy(fout0, acc.at[didxf.at[b0]], ssem0, add=True)

            @pl.when(p > 0)
            def _():
                pltpu.make_async_copy(fout1, acc.at[didxf.at[0]],
                                      ssem1).wait()
            pltpu.make_async_copy(gsrc(b1), featb1, gsem1).wait()
            _mul_block(featb1, fout1, wbuff, b1 * EB)
            pltpu.async_copy(fout1, acc.at[didxf.at[b1]], ssem1, add=True)
            return 0
        lax.fori_loop(0, E32 // EB // 2, pair, 0)
        pltpu.make_async_copy(fout0, acc.at[didxf.at[0]], ssem0).wait()
        pltpu.make_async_copy(fout1, acc.at[didxf.at[0]], ssem1).wait()
        return 0
    lax.fori_loop(0, 2, half, 0)


def _gat1_body(as1t, ad1t, srcp, dstp2, zh64, zh16, *rest):
    h1c = rest[0:NCH1]
    w1f, den1 = rest[NCH1], rest[NCH1 + 1]
    out1 = rest[NCH1 + 2:NCH1 + 2 + NCH1]
    (ash, adh, rbuf0, rbuf1, sidxf, didxf, wbuff, featb0, featb1,
     fout0, fout1, acc, dacc, gsem0, gsem1, ssem0, ssem1) \
        = rest[NCH1 + 2 + NCH1:]
    cid = lax.axis_index("c")
    sid = lax.axis_index("s")
    row0 = sid * ROWS_T

    pltpu.sync_copy(zh16.at[pl.ds(0, EB)], rbuf0)
    pltpu.sync_copy(zh16.at[pl.ds(0, EB)], rbuf1)
    pltpu.sync_copy(zh16, dacc.at[pl.ds(row0, ROWS_T)])
    plsc.subcore_barrier()

    for hs in range(4):
        @pl.when(cid * 2 + sid // 8 == hs)
        def _():
            pltpu.sync_copy(as1t.at[hs], ash)
            pltpu.sync_copy(ad1t.at[hs], adh)
    r = sid % 8
    g = sid // 8
    hq = cid * 2 + g
    base = r * E8

    def wquarter(hh, _):
        e0 = base + hh * E32
        _w_range(ash, adh, srcp, dstp2, sidxf, didxf, wbuff, rbuf0, rbuf1,
                 gsem0, gsem1, dacc, hq, e0, E32 // EB)
        pltpu.sync_copy(wbuff.at[pl.ds(0, E32)],
                        w1f.at[pl.ds(hq * EPAD + e0, E32)])
        return 0
    lax.fori_loop(0, 4, wquarter, 0)
    plsc.subcore_barrier()

    pltpu.sync_copy(dacc.at[pl.ds(row0, ROWS_T)],
                    den1.at[pl.ds(cid * NPAD + row0, ROWS_T)])

    for step in range(8):
        pltpu.sync_copy(zh64, acc.at[pl.ds(row0, ROWS_T)])
        plsc.subcore_barrier()
        for cc in range(2):
            c = cc * 8 + step
            h = c // 4

            @pl.when(cid == cc)
            def _(c=c, h=h):
                _f_step(h1c[c], w1f, srcp, dstp2, sidxf, didxf, wbuff,
                        featb0, featb1, fout0, fout1, gsem0, gsem1,
                        ssem0, ssem1, acc, sid, h * EPAD)
        plsc.subcore_barrier()
        for cc in range(2):
            c = cc * 8 + step

            @pl.when(cid == cc)
            def _(c=c):
                pltpu.sync_copy(acc.at[pl.ds(row0, ROWS_T)],
                                out1[c].at[pl.ds(row0, ROWS_T)])
        plsc.subcore_barrier()


def _run_gat1(as1t, ad1t, srcp, dstp2, zh64, zh16, h1c):
    kern = pl.kernel(
        _gat1_body,
        out_type=[jax.ShapeDtypeStruct((4 * EPAD,), jnp.float32),
                  jax.ShapeDtypeStruct((2 * NPAD, 16), jnp.float32)]
        + [jax.ShapeDtypeStruct((NPAD, 64), jnp.float32)] * NCH1,
        mesh=_mesh(),
        compiler_params=pltpu.CompilerParams(needs_layout_passes=False, use_tc_tiling_on_sc=False),
        scratch_types=[
            pltpu.VMEM((NPAD,), jnp.float32),
            pltpu.VMEM((NPAD,), jnp.float32),
            pltpu.VMEM((EB, 16), jnp.float32),
            pltpu.VMEM((EB, 16), jnp.float32),
            pltpu.VMEM((E32,), jnp.int32),
            pltpu.VMEM((E32 // EB, EB), jnp.int32),
            pltpu.VMEM((E32,), jnp.float32),
            pltpu.VMEM((EB, 64), jnp.bfloat16),
            pltpu.VMEM((EB, 64), jnp.bfloat16),
            pltpu.VMEM((EB, 64), jnp.float32),
            pltpu.VMEM((EB, 64), jnp.float32),
            pltpu.VMEM_SHARED((NPAD, 64), jnp.float32),
            pltpu.VMEM_SHARED((NPAD, 16), jnp.float32),
            pltpu.SemaphoreType.DMA,
            pltpu.SemaphoreType.DMA,
            pltpu.SemaphoreType.DMA,
            pltpu.SemaphoreType.DMA,
        ],
    )
    return kern(as1t, ad1t, srcp, dstp2, zh64, zh16, *h1c)


def _mid_body(*refs):
    o1 = refs[0:NCH1]
    dnb, b1r, w2, a2s, a2d = refs[NCH1:NCH1 + 5]
    outs = refs[NCH1 + 5:]
    h2c = outs[0:NCH2]
    as2, ad2 = outs[NCH2], outs[NCH2 + 1]
    i = pl.program_id(0)
    cols = []
    for c in range(NCH1):
        dnh = dnb[c // 4, :][:, None]
        xc = o1[c][...] / (dnh + 1e-16) + b1r[0, c * 64:(c + 1) * 64][None, :]
        cols.append(_elu(xc))
    x2 = jnp.concatenate(cols, axis=1)
    rows = i * 256 + lax.broadcasted_iota(jnp.int32, (256, 1), 0)
    x2 = jnp.where(rows < N, x2, 0.0)
    h2 = jnp.dot(x2, w2[...], preferred_element_type=jnp.float32)
    for c in range(NCH2):
        h2c[c][...] = h2[:, c * 64:(c + 1) * 64].astype(jnp.bfloat16)
    as2[...] = jnp.dot(h2, a2s[...], preferred_element_type=jnp.float32)
    ad2[...] = jnp.dot(h2, a2d[...], preferred_element_type=jnp.float32)


def _run_mid(out1, den1s, b1, w2, att_src2, att_dst2):
    full = lambda shape: pl.BlockSpec(shape, lambda i: (0,) * len(shape))
    outs = pl.pallas_call(
        _mid_body,
        grid=(NB,),
        in_specs=[pl.BlockSpec((256, 64), lambda i: (i, 0))] * NCH1
        + [pl.BlockSpec((4, 256), lambda i: (0, i)),
           full((1, 4 * HID)), full((4 * HID, HID)),
           full((HID, 1)), full((HID, 1))],
        out_specs=[pl.BlockSpec((256, 64), lambda i: (i, 0))] * NCH2
        + [pl.BlockSpec((256, 1), lambda i: (i, 0))] * 2,
        out_shape=[jax.ShapeDtypeStruct((NPAD, 64), jnp.bfloat16)] * NCH2
        + [jax.ShapeDtypeStruct((NPAD, 1), jnp.float32)] * 2,
    )(*out1, den1s, b1.reshape(1, 4 * HID), w2, att_src2, att_dst2)
    return outs[0:NCH2], outs[NCH2], outs[NCH2 + 1]


def _gat2_body(as2f, ad2f, srcp, dstp2, zh64, zh16, h2c0, h2c1, h2c2, h2c3,
               w2f, den2, o2a, o2b, o2c, o2d,
               ash, adh, rbuf0, rbuf1, sidxf, didxf, wbuff,
               featb0, featb1, fout0, fout1, acc, dacc,
               gsem0, gsem1, ssem0, ssem1):
    h2c = (h2c0, h2c1, h2c2, h2c3)
    out2 = (o2a, o2b, o2c, o2d)
    cid = lax.axis_index("c")
    sid = lax.axis_index("s")
    row0 = sid * ROWS_T

    pltpu.sync_copy(zh16.at[pl.ds(0, EB)], rbuf0)
    pltpu.sync_copy(zh16.at[pl.ds(0, EB)], rbuf1)
    pltpu.sync_copy(zh16, dacc.at[pl.ds(row0, ROWS_T)])
    plsc.subcore_barrier()

    pltpu.sync_copy(as2f, ash)
    pltpu.sync_copy(ad2f, adh)
    r32 = cid * 16 + sid
    base = r32 * E32
    _w_range(ash, adh, srcp, dstp2, sidxf, didxf, wbuff, rbuf0, rbuf1,
             gsem0, gsem1, dacc, 0, base, E32 // EB)
    pltpu.sync_copy(wbuff.at[pl.ds(0, E32)], w2f.at[pl.ds(base, E32)])
    plsc.subcore_barrier()

    pltpu.sync_copy(dacc.at[pl.ds(row0, ROWS_T)],
                    den2.at[pl.ds(cid * NPAD + row0, ROWS_T)])

    for step in range(2):
        pltpu.sync_copy(zh64, acc.at[pl.ds(row0, ROWS_T)])
        plsc.subcore_barrier()
        for cc in range(2):
            c = cc * 2 + step

            @pl.when(cid == cc)
            def _(c=c):
                _f_step(h2c[c], w2f, srcp, dstp2, sidxf, didxf, wbuff,
                        featb0, featb1, fout0, fout1, gsem0, gsem1,
                        ssem0, ssem1, acc, sid, 0)
        plsc.subcore_barrier()
        for cc in range(2):
            c = cc * 2 + step

            @pl.when(cid == cc)
            def _(c=c):
                pltpu.sync_copy(acc.at[pl.ds(row0, ROWS_T)],
                                out2[c].at[pl.ds(row0, ROWS_T)])
        plsc.subcore_barrier()


def _run_gat2(as2f, ad2f, srcp, dstp2, zh64, zh16, h2c):
    kern = pl.kernel(
        _gat2_body,
        out_type=[jax.ShapeDtypeStruct((EPAD,), jnp.float32),
                  jax.ShapeDtypeStruct((2 * NPAD, 16), jnp.float32)]
        + [jax.ShapeDtypeStruct((NPAD, 64), jnp.float32)] * NCH2,
        mesh=_mesh(),
        compiler_params=pltpu.CompilerParams(needs_layout_passes=False, use_tc_tiling_on_sc=False),
        scratch_types=[
            pltpu.VMEM((NPAD,), jnp.float32),
            pltpu.VMEM((NPAD,), jnp.float32),
            pltpu.VMEM((EB, 16), jnp.float32),
            pltpu.VMEM((EB, 16), jnp.float32),
            pltpu.VMEM((E32,), jnp.int32),
            pltpu.VMEM((E32 // EB, EB), jnp.int32),
            pltpu.VMEM((E32,), jnp.float32),
            pltpu.VMEM((EB, 64), jnp.bfloat16),
            pltpu.VMEM((EB, 64), jnp.bfloat16),
            pltpu.VMEM((EB, 64), jnp.float32),
            pltpu.VMEM((EB, 64), jnp.float32),
            pltpu.VMEM_SHARED((NPAD, 64), jnp.float32),
            pltpu.VMEM_SHARED((NPAD, 16), jnp.float32),
            pltpu.SemaphoreType.DMA,
            pltpu.SemaphoreType.DMA,
            pltpu.SemaphoreType.DMA,
            pltpu.SemaphoreType.DMA,
        ],
    )
    return kern(as2f, ad2f, srcp, dstp2, zh64, zh16, *h2c)


def _head_body(*refs):
    o2 = refs[0:NCH2]
    dnb, b2r, bat, stf, wc1, bc1, lg, lb, wc2, bc2 = refs[NCH2:NCH2 + 10]
    out, pooled, cnt = refs[NCH2 + 10:]
    i = pl.program_id(0)

    @pl.when(i == 0)
    def _():
        pooled[...] = jnp.zeros((B, HID), jnp.float32)
        cnt[...] = jnp.zeros((B, 128), jnp.float32)

    d2 = (dnb[0, :] + dnb[1, :])[:, None]
    x3 = jnp.concatenate([o2[c][...] for c in range(NCH2)], axis=1)
    x3 = _elu(x3 / (d2 + 1e-16) + b2r[...])
    onehot = (bat[0, 0, :][None, :] ==
              lax.broadcasted_iota(jnp.int32, (B, 256), 0)).astype(jnp.float32)
    pooled[...] += jnp.dot(onehot, x3, preferred_element_type=jnp.float32)
    s = jnp.sum(onehot, axis=1, keepdims=True)
    cnt[...] += jnp.broadcast_to(s, (B, 128))

    @pl.when(i == NB - 1)
    def _():
        counts = cnt[:, 0:1]
        pm = pooled[...] / jnp.maximum(counts, 1.0)
        z = (jnp.dot(pm, wc1[0:HID, :], preferred_element_type=jnp.float32)
             + jnp.dot(stf[...], wc1[HID:HID + STATIC, :],
                       preferred_element_type=jnp.float32) + bc1[...])
        mu = jnp.mean(z, axis=1, keepdims=True)
        var = jnp.mean((z - mu) * (z - mu), axis=1, keepdims=True)
        z = (z - mu) / jnp.sqrt(var + 1e-5) * lg[...] + lb[...]
        z = jnp.maximum(z, 0.0)
        out[...] = jnp.dot(z, wc2[...],
                           preferred_element_type=jnp.float32) + bc2[...]


def _run_head(out2, den2s, b2, bat3, stf, wc1, bc1, lg, lb, wc2, bc2):
    full = lambda shape: pl.BlockSpec(shape, lambda i: (0,) * len(shape))
    return pl.pallas_call(
        _head_body,
        grid=(NB,),
        in_specs=[pl.BlockSpec((256, 64), lambda i: (i, 0))] * NCH2
        + [pl.BlockSpec((2, 256), lambda i: (0, i)),
           full((1, HID)),
           pl.BlockSpec((1, 1, 256), lambda i: (i, 0, 0)),
           full((B, STATIC)), full((HID + STATIC, 128)), full((1, 128)),
           full((1, 128)), full((1, 128)), full((128, NCLS)),
           full((1, NCLS))],
        out_specs=pl.BlockSpec((B, NCLS), lambda i: (0, 0)),
        out_shape=jax.ShapeDtypeStruct((B, NCLS), jnp.float32),
        scratch_shapes=[pltpu.VMEM((B, HID), jnp.float32),
                        pltpu.VMEM((B, 128), jnp.float32)],
    )(*out2, den2s, b2.reshape(1, HID), bat3, stf, wc1,
      bc1.reshape(1, 128), lg.reshape(1, 128), lb.reshape(1, 128),
      wc2, bc2.reshape(1, NCLS))


def kernel(node_features, edge_index, batch, static_features, W_proj, b_proj,
           ln1_g, ln1_b, W1, att_src1, att_dst1, b1, W2, att_src2, att_dst2,
           b2, Wc1, bc1, lnc_g, lnc_b, Wc2, bc2):
    f32 = jnp.float32
    nfp = jnp.pad(node_features, ((0, NPAD - N), (0, 0)))
    loop = jnp.arange(N, dtype=jnp.int32)
    npadidx = jnp.full((EPAD - E - N,), NPAD - 1, jnp.int32)
    srcp = jnp.concatenate([edge_index[0].astype(jnp.int32), loop, npadidx])
    dstp = jnp.concatenate([edge_index[1].astype(jnp.int32), loop, npadidx])
    dstp2 = dstp.reshape(EPAD // EB, EB)
    batp = jnp.concatenate([batch.astype(jnp.int32),
                            jnp.full((NPAD - N,), B, jnp.int32)])
    bat3 = batp.reshape(NB, 1, 256)
    zh64 = jnp.zeros((ROWS_T, 64), f32)
    zh16 = jnp.zeros((ROWS_T, 16), f32)

    eye4 = jnp.eye(HEADS, dtype=f32)
    asm = (att_src1[:, :, None] * eye4[:, None, :]).reshape(HEADS * HID, HEADS)
    adm = (att_dst1[:, :, None] * eye4[:, None, :]).reshape(HEADS * HID, HEADS)

    g32 = jnp.arange(32, dtype=jnp.int32).reshape(2, 16).T.reshape(-1)
    perm1 = (jnp.arange(0, HEADS * HID, 32, dtype=jnp.int32)[:, None]
             + g32[None, :]).reshape(-1)
    perm2 = (jnp.arange(0, HID, 32, dtype=jnp.int32)[:, None]
             + g32[None, :]).reshape(-1)
    W1p = W1[:, perm1]
    asmp = asm[perm1]
    admp = adm[perm1]
    W2p = W2[:, perm2]
    a2sp = att_src2.reshape(HID, 1)[perm2]
    a2dp = att_dst2.reshape(HID, 1)[perm2]

    h1c, as1, ad1 = _run_proj(nfp, W_proj, b_proj, ln1_g, ln1_b, W1p, asmp,
                              admp)
    as1t = as1.T.reshape(HEADS, NPAD)
    ad1t = ad1.T.reshape(HEADS, NPAD)

    g1 = _run_gat1(as1t, ad1t, srcp, dstp2, zh64, zh16, h1c)
    den1f, out1 = g1[1], g1[2:2 + NCH1]
    den1r = den1f.reshape(2, NPAD, 16)
    den1s = jnp.stack([den1r[0, :, 0], den1r[0, :, 1],
                       den1r[1, :, 2], den1r[1, :, 3]])

    h2c, as2, ad2 = _run_mid(out1, den1s, b1, W2p, a2sp, a2dp)
    as2f = as2[:, 0]
    ad2f = ad2[:, 0]

    g2 = _run_gat2(as2f, ad2f, srcp, dstp2, zh64, zh16, h2c)
    den2f, out2 = g2[1], g2[2:2 + NCH2]
    den2s = den2f[:, 0].reshape(2, NPAD)

    return _run_head(out2, den2s, b2, bat3, static_features, Wc1, bc1,
                     lnc_g, lnc_b, Wc2, bc2)


# speedup vs baseline: 13.1184x; 1.1288x over previous
import functools
import jax
import jax.numpy as jnp
from jax import lax
from jax.experimental import pallas as pl
from jax.experimental.pallas import tpu as pltpu
from jax.experimental.pallas import tpu_sc as plsc

N = 10000
E = 160000
B = 16
IN_DIM = 768
HID = 256
HEADS = 4
NCLS = 5
STATIC = 12

NPAD = 10240
NB = NPAD // 256
EPAD = 172032
EB = 128
E8 = EPAD // 8
E16 = EPAD // 16
E32 = EPAD // 32
ROWS_T = NPAD // 16
NCH1 = 16
NCH2 = 4


@functools.cache
def _mesh():
    return plsc.VectorSubcoreMesh(
        core_axis_name="c", subcore_axis_name="s", num_cores=2,
        num_subcores=16)


def _elu(x):
    return jnp.where(x > 0, x, jnp.exp(jnp.minimum(x, 0.0)) - 1.0)


def _proj_body(nf, wp, bp, g1, b1ln, w1, asm, adm, *outs):
    h1c = outs[0:NCH1]
    as1, ad1 = outs[NCH1], outs[NCH1 + 1]
    i = pl.program_id(0)
    x = jnp.dot(nf[...], wp[...], preferred_element_type=jnp.float32) + bp[...]
    mu = jnp.mean(x, axis=1, keepdims=True)
    var = jnp.mean((x - mu) * (x - mu), axis=1, keepdims=True)
    x = (x - mu) / jnp.sqrt(var + 1e-5) * g1[...] + b1ln[...]
    x = jnp.maximum(x, 0.0)
    rows = i * 256 + lax.broadcasted_iota(jnp.int32, (256, 1), 0)
    x = jnp.where(rows < N, x, 0.0)
    h1 = jnp.dot(x, w1[...], preferred_element_type=jnp.float32)
    for c in range(NCH1):
        h1c[c][...] = h1[:, c * 64:(c + 1) * 64]
    as1[...] = jnp.dot(h1, asm[...], preferred_element_type=jnp.float32)
    ad1[...] = jnp.dot(h1, adm[...], preferred_element_type=jnp.float32)


def _run_proj(nfp, w_proj, b_proj, ln1_g, ln1_b, w1, asm, adm):
    full = lambda shape: pl.BlockSpec(shape, lambda i: (0,) * len(shape))
    outs = pl.pallas_call(
        _proj_body,
        grid=(NB,),
        in_specs=[
            pl.BlockSpec((256, IN_DIM), lambda i: (i, 0)),
            full((IN_DIM, HID)), full((1, HID)), full((1, HID)),
            full((1, HID)), full((HID, 4 * HID)),
            full((4 * HID, 4)), full((4 * HID, 4)),
        ],
        out_specs=[pl.BlockSpec((256, 64), lambda i: (i, 0))] * NCH1
        + [pl.BlockSpec((256, 4), lambda i: (i, 0))] * 2,
        out_shape=[jax.ShapeDtypeStruct((NPAD, 64), jnp.float32)] * NCH1
        + [jax.ShapeDtypeStruct((NPAD, 4), jnp.float32)] * 2,
    )(nfp, w_proj, b_proj.reshape(1, HID), ln1_g.reshape(1, HID),
      ln1_b.reshape(1, HID), w1, asm, adm)
    return outs[0:NCH1], outs[NCH1], outs[NCH1 + 1]


def _w_inner(ash, adh, sidxf, didxf, wbuff, rbuf, hcol, b):
    lane = lax.broadcasted_iota(jnp.int32, (16,), 0)
    hvec = jnp.full((16,), hcol, jnp.int32)

    def body(i, _):
        sl = pl.ds(b * EB + i * 16, 16)
        s = sidxf[sl]
        d = didxf[b, pl.ds(i * 16, 16)]
        av = plsc.load_gather(ash, [s]) + plsc.load_gather(adh, [d])
        av = jnp.where(av > 0, av, 0.2 * av)
        w = jnp.exp(av)
        wbuff[sl] = w
        plsc.store_scatter(rbuf, [i * 16 + lane, hvec], w)
        return 0
    lax.fori_loop(0, EB // 16, body, 0)


def _w_range(ash, adh, srcp, dstp2, sidxf, didxf, wbuff, rbuf0, rbuf1,
             wsem0, wsem1, dacc, hcol, e0, nblk):
    ne = nblk * EB
    pltpu.sync_copy(srcp.at[pl.ds(e0, ne)], sidxf.at[pl.ds(0, ne)])
    pltpu.sync_copy(dstp2.at[pl.ds(e0 // EB, nblk)], didxf.at[pl.ds(0, nblk)])

    def pair(p, _):
        b0 = 2 * p
        b1 = b0 + 1

        @pl.when(p > 0)
        def _():
            pltpu.make_async_copy(rbuf0, dacc.at[didxf.at[0]], wsem0).wait()
        _w_inner(ash, adh, sidxf, didxf, wbuff, rbuf0, hcol, b0)
        pltpu.async_copy(rbuf0, dacc.at[didxf.at[b0]], wsem0, add=True)

        @pl.when(p > 0)
        def _():
            pltpu.make_async_copy(rbuf1, dacc.at[didxf.at[0]], wsem1).wait()
        _w_inner(ash, adh, sidxf, didxf, wbuff, rbuf1, hcol, b1)
        pltpu.async_copy(rbuf1, dacc.at[didxf.at[b1]], wsem1, add=True)
        return 0
    lax.fori_loop(0, nblk // 2, pair, 0)
    pltpu.make_async_copy(rbuf0, dacc.at[didxf.at[0]], wsem0).wait()
    pltpu.make_async_copy(rbuf1, dacc.at[didxf.at[0]], wsem1).wait()


def _mul_block(feat, wbuff, boff):
    def mul(i, _):
        wv16 = wbuff[pl.ds(boff + i * 16, 16)]
        for j in range(16):
            k = i * 16 + j
            wv = jnp.full((16,), wv16[j], jnp.float32)
            for m in range(4):
                sl = pl.ds(m * 16, 16)
                feat[k, sl] = feat[k, sl] * wv
        return 0
    lax.fori_loop(0, EB // 16, mul, 0)


def _f_step(h1ref, wf, srcp, dstp2, sidxf, didxf, wbuff, feat0, feat1,
            gsem0, gsem1, ssem0, ssem1, acc, sid, woff_base):
    ebase0 = sid * E16
    pltpu.sync_copy(srcp.at[pl.ds(ebase0, E16)], sidxf)
    pltpu.sync_copy(wf.at[pl.ds(woff_base + ebase0, E16)], wbuff)
    pltpu.sync_copy(dstp2.at[pl.ds(sid * (E16 // EB), E16 // EB)], didxf)

    def gsrc(b):
        return h1ref.at[sidxf.at[pl.ds(b * EB, EB)]]

    def pair(p, _):
        b0 = 2 * p
        b1 = b0 + 1

        @pl.when(p > 0)
        def _():
            pltpu.make_async_copy(feat0, acc.at[didxf.at[0]], ssem0).wait()
        pltpu.async_copy(gsrc(b0), feat0, gsem0)

        @pl.when(p > 0)
        def _():
            pltpu.make_async_copy(feat1, acc.at[didxf.at[0]], ssem1).wait()
        pltpu.async_copy(gsrc(b1), feat1, gsem1)

        pltpu.make_async_copy(gsrc(b0), feat0, gsem0).wait()
        _mul_block(feat0, wbuff, b0 * EB)
        pltpu.async_copy(feat0, acc.at[didxf.at[b0]], ssem0, add=True)

        pltpu.make_async_copy(gsrc(b1), feat1, gsem1).wait()
        _mul_block(feat1, wbuff, b1 * EB)
        pltpu.async_copy(feat1, acc.at[didxf.at[b1]], ssem1, add=True)
        return 0
    lax.fori_loop(0, E16 // EB // 2, pair, 0)
    pltpu.make_async_copy(feat0, acc.at[didxf.at[0]], ssem0).wait()
    pltpu.make_async_copy(feat1, acc.at[didxf.at[0]], ssem1).wait()


def _gat1_body(as1t, ad1t, srcp, dstp2, zh64, zh16, *rest):
    h1c = rest[0:NCH1]
    w1f, den1 = rest[NCH1], rest[NCH1 + 1]
    out1 = rest[NCH1 + 2:NCH1 + 2 + NCH1]
    (ash, adh, rbuf0, rbuf1, sidxf, didxf, wbuff, feat0, feat1,
     acc, dacc, gsem0, gsem1, ssem0, ssem1) = rest[NCH1 + 2 + NCH1:]
    cid = lax.axis_index("c")
    sid = lax.axis_index("s")
    row0 = sid * ROWS_T

    pltpu.sync_copy(zh16.at[pl.ds(0, EB)], rbuf0)
    pltpu.sync_copy(zh16.at[pl.ds(0, EB)], rbuf1)
    pltpu.sync_copy(zh16, dacc.at[pl.ds(row0, ROWS_T)])
    plsc.subcore_barrier()

    for hs in range(4):
        @pl.when(cid * 2 + sid // 8 == hs)
        def _():
            pltpu.sync_copy(as1t.at[hs], ash)
            pltpu.sync_copy(ad1t.at[hs], adh)
    r = sid % 8
    g = sid // 8
    hq = cid * 2 + g
    base = r * E8
    for hh in range(2):
        e0 = base + hh * E16
        _w_range(ash, adh, srcp, dstp2, sidxf, didxf, wbuff, rbuf0, rbuf1,
                 gsem0, gsem1, dacc, hq, e0, E16 // EB)
        pltpu.sync_copy(wbuff, w1f.at[pl.ds(hq * EPAD + e0, E16)])
    plsc.subcore_barrier()

    pltpu.sync_copy(dacc.at[pl.ds(row0, ROWS_T)],
                    den1.at[pl.ds(cid * NPAD + row0, ROWS_T)])

    for step in range(8):
        pltpu.sync_copy(zh64, acc.at[pl.ds(row0, ROWS_T)])
        plsc.subcore_barrier()
        for cc in range(2):
            c = cc * 8 + step
            h = c // 4

            @pl.when(cid == cc)
            def _(c=c, h=h):
                _f_step(h1c[c], w1f, srcp, dstp2, sidxf, didxf, wbuff,
                        feat0, feat1, gsem0, gsem1, ssem0, ssem1, acc, sid,
                        h * EPAD)
        plsc.subcore_barrier()
        for cc in range(2):
            c = cc * 8 + step

            @pl.when(cid == cc)
            def _(c=c):
                pltpu.sync_copy(acc.at[pl.ds(row0, ROWS_T)],
                                out1[c].at[pl.ds(row0, ROWS_T)])
        plsc.subcore_barrier()


def _run_gat1(as1t, ad1t, srcp, dstp2, zh64, zh16, h1c):
    kern = pl.kernel(
        _gat1_body,
        out_type=[jax.ShapeDtypeStruct((4 * EPAD,), jnp.float32),
                  jax.ShapeDtypeStruct((2 * NPAD, 16), jnp.float32)]
        + [jax.ShapeDtypeStruct((NPAD, 64), jnp.float32)] * NCH1,
        mesh=_mesh(),
        compiler_params=pltpu.CompilerParams(needs_layout_passes=False, use_tc_tiling_on_sc=False),
        scratch_types=[
            pltpu.VMEM((NPAD,), jnp.float32),
            pltpu.VMEM((NPAD,), jnp.float32),
            pltpu.VMEM((EB, 16), jnp.float32),
            pltpu.VMEM((EB, 16), jnp.float32),
            pltpu.VMEM((E16,), jnp.int32),
            pltpu.VMEM((E16 // EB, EB), jnp.int32),
            pltpu.VMEM((E16,), jnp.float32),
            pltpu.VMEM((EB, 64), jnp.float32),
            pltpu.VMEM((EB, 64), jnp.float32),
            pltpu.VMEM_SHARED((NPAD, 64), jnp.float32),
            pltpu.VMEM_SHARED((NPAD, 16), jnp.float32),
            pltpu.SemaphoreType.DMA,
            pltpu.SemaphoreType.DMA,
            pltpu.SemaphoreType.DMA,
            pltpu.SemaphoreType.DMA,
        ],
    )
    return kern(as1t, ad1t, srcp, dstp2, zh64, zh16, *h1c)


def _mid_body(*refs):
    o1 = refs[0:NCH1]
    dnb, b1r, w2, a2s, a2d = refs[NCH1:NCH1 + 5]
    outs = refs[NCH1 + 5:]
    h2c = outs[0:NCH2]
    as2, ad2 = outs[NCH2], outs[NCH2 + 1]
    i = pl.program_id(0)
    cols = []
    for c in range(NCH1):
        dnh = dnb[c // 4, :][:, None]
        xc = o1[c][...] / (dnh + 1e-16) + b1r[0, c * 64:(c + 1) * 64][None, :]
        cols.append(_elu(xc))
    x2 = jnp.concatenate(cols, axis=1)
    rows = i * 256 + lax.broadcasted_iota(jnp.int32, (256, 1), 0)
    x2 = jnp.where(rows < N, x2, 0.0)
    h2 = jnp.dot(x2, w2[...], preferred_element_type=jnp.float32)
    for c in range(NCH2):
        h2c[c][...] = h2[:, c * 64:(c + 1) * 64]
    as2[...] = jnp.dot(h2, a2s[...], preferred_element_type=jnp.float32)
    ad2[...] = jnp.dot(h2, a2d[...], preferred_element_type=jnp.float32)


def _run_mid(out1, den1s, b1, w2, att_src2, att_dst2):
    full = lambda shape: pl.BlockSpec(shape, lambda i: (0,) * len(shape))
    outs = pl.pallas_call(
        _mid_body,
        grid=(NB,),
        in_specs=[pl.BlockSpec((256, 64), lambda i: (i, 0))] * NCH1
        + [pl.BlockSpec((4, 256), lambda i: (0, i)),
           full((1, 4 * HID)), full((4 * HID, HID)),
           full((HID, 1)), full((HID, 1))],
        out_specs=[pl.BlockSpec((256, 64), lambda i: (i, 0))] * NCH2
        + [pl.BlockSpec((256, 1), lambda i: (i, 0))] * 2,
        out_shape=[jax.ShapeDtypeStruct((NPAD, 64), jnp.float32)] * NCH2
        + [jax.ShapeDtypeStruct((NPAD, 1), jnp.float32)] * 2,
    )(*out1, den1s, b1.reshape(1, 4 * HID), w2,
      att_src2.reshape(HID, 1), att_dst2.reshape(HID, 1))
    return outs[0:NCH2], outs[NCH2], outs[NCH2 + 1]


def _gat2_body(as2f, ad2f, srcp, dstp2, zh64, zh16, h2c0, h2c1, h2c2, h2c3,
               w2f, den2, o2a, o2b, o2c, o2d,
               ash, adh, rbuf0, rbuf1, sidxf, didxf, wbuff,
               feat0, feat1, acc, dacc, gsem0, gsem1, ssem0, ssem1):
    h2c = (h2c0, h2c1, h2c2, h2c3)
    out2 = (o2a, o2b, o2c, o2d)
    cid = lax.axis_index("c")
    sid = lax.axis_index("s")
    row0 = sid * ROWS_T

    pltpu.sync_copy(zh16.at[pl.ds(0, EB)], rbuf0)
    pltpu.sync_copy(zh16.at[pl.ds(0, EB)], rbuf1)
    pltpu.sync_copy(zh16, dacc.at[pl.ds(row0, ROWS_T)])
    plsc.subcore_barrier()

    pltpu.sync_copy(as2f, ash)
    pltpu.sync_copy(ad2f, adh)
    r32 = cid * 16 + sid
    base = r32 * E32
    _w_range(ash, adh, srcp, dstp2, sidxf, didxf, wbuff, rbuf0, rbuf1,
             gsem0, gsem1, dacc, 0, base, E32 // EB)
    pltpu.sync_copy(wbuff.at[pl.ds(0, E32)], w2f.at[pl.ds(base, E32)])
    plsc.subcore_barrier()

    pltpu.sync_copy(dacc.at[pl.ds(row0, ROWS_T)],
                    den2.at[pl.ds(cid * NPAD + row0, ROWS_T)])

    for step in range(2):
        pltpu.sync_copy(zh64, acc.at[pl.ds(row0, ROWS_T)])
        plsc.subcore_barrier()
        for cc in range(2):
            c = cc * 2 + step

            @pl.when(cid == cc)
            def _(c=c):
                _f_step(h2c[c], w2f, srcp, dstp2, sidxf, didxf, wbuff,
                        feat0, feat1, gsem0, gsem1, ssem0, ssem1, acc, sid, 0)
        plsc.subcore_barrier()
        for cc in range(2):
            c = cc * 2 + step

            @pl.when(cid == cc)
            def _(c=c):
                pltpu.sync_copy(acc.at[pl.ds(row0, ROWS_T)],
                                out2[c].at[pl.ds(row0, ROWS_T)])
        plsc.subcore_barrier()


def _run_gat2(as2f, ad2f, srcp, dstp2, zh64, zh16, h2c):
    kern = pl.kernel(
        _gat2_body,
        out_type=[jax.ShapeDtypeStruct((EPAD,), jnp.float32),
                  jax.ShapeDtypeStruct((2 * NPAD, 16), jnp.float32)]
        + [jax.ShapeDtypeStruct((NPAD, 64), jnp.float32)] * NCH2,
        mesh=_mesh(),
        compiler_params=pltpu.CompilerParams(needs_layout_passes=False, use_tc_tiling_on_sc=False),
        scratch_types=[
            pltpu.VMEM((NPAD,), jnp.float32),
            pltpu.VMEM((NPAD,), jnp.float32),
            pltpu.VMEM((EB, 16), jnp.float32),
            pltpu.VMEM((EB, 16), jnp.float32),
            pltpu.VMEM((E16,), jnp.int32),
            pltpu.VMEM((E16 // EB, EB), jnp.int32),
            pltpu.VMEM((E16,), jnp.float32),
            pltpu.VMEM((EB, 64), jnp.float32),
            pltpu.VMEM((EB, 64), jnp.float32),
            pltpu.VMEM_SHARED((NPAD, 64), jnp.float32),
            pltpu.VMEM_SHARED((NPAD, 16), jnp.float32),
            pltpu.SemaphoreType.DMA,
            pltpu.SemaphoreType.DMA,
            pltpu.SemaphoreType.DMA,
            pltpu.SemaphoreType.DMA,
        ],
    )
    return kern(as2f, ad2f, srcp, dstp2, zh64, zh16, *h2c)


def _head_body(*refs):
    o2 = refs[0:NCH2]
    dnb, b2r, bat, stf, wc1, bc1, lg, lb, wc2, bc2 = refs[NCH2:NCH2 + 10]
    out, pooled, cnt = refs[NCH2 + 10:]
    i = pl.program_id(0)

    @pl.when(i == 0)
    def _():
        pooled[...] = jnp.zeros((B, HID), jnp.float32)
        cnt[...] = jnp.zeros((B, 128), jnp.float32)

    d2 = (dnb[0, :] + dnb[1, :])[:, None]
    x3 = jnp.concatenate([o2[c][...] for c in range(NCH2)], axis=1)
    x3 = _elu(x3 / (d2 + 1e-16) + b2r[...])
    onehot = (bat[0, 0, :][None, :] ==
              lax.broadcasted_iota(jnp.int32, (B, 256), 0)).astype(jnp.float32)
    pooled[...] += jnp.dot(onehot, x3, preferred_element_type=jnp.float32)
    s = jnp.sum(onehot, axis=1, keepdims=True)
    cnt[...] += jnp.broadcast_to(s, (B, 128))

    @pl.when(i == NB - 1)
    def _():
        counts = cnt[:, 0:1]
        pm = pooled[...] / jnp.maximum(counts, 1.0)
        z = (jnp.dot(pm, wc1[0:HID, :], preferred_element_type=jnp.float32)
             + jnp.dot(stf[...], wc1[HID:HID + STATIC, :],
                       preferred_element_type=jnp.float32) + bc1[...])
        mu = jnp.mean(z, axis=1, keepdims=True)
        var = jnp.mean((z - mu) * (z - mu), axis=1, keepdims=True)
        z = (z - mu) / jnp.sqrt(var + 1e-5) * lg[...] + lb[...]
        z = jnp.maximum(z, 0.0)
        out[...] = jnp.dot(z, wc2[...],
                           preferred_element_type=jnp.float32) + bc2[...]


def _run_head(out2, den2s, b2, bat3, stf, wc1, bc1, lg, lb, wc2, bc2):
    full = lambda shape: pl.BlockSpec(shape, lambda i: (0,) * len(shape))
    return pl.pallas_call(
        _head_body,
        grid=(NB,),
        in_specs=[pl.BlockSpec((256, 64), lambda i: (i, 0))] * NCH2
        + [pl.BlockSpec((2, 256), lambda i: (0, i)),
           full((1, HID)),
           pl.BlockSpec((1, 1, 256), lambda i: (i, 0, 0)),
           full((B, STATIC)), full((HID + STATIC, 128)), full((1, 128)),
           full((1, 128)), full((1, 128)), full((128, NCLS)),
           full((1, NCLS))],
        out_specs=pl.BlockSpec((B, NCLS), lambda i: (0, 0)),
        out_shape=jax.ShapeDtypeStruct((B, NCLS), jnp.float32),
        scratch_shapes=[pltpu.VMEM((B, HID), jnp.float32),
                        pltpu.VMEM((B, 128), jnp.float32)],
    )(*out2, den2s, b2.reshape(1, HID), bat3, stf, wc1,
      bc1.reshape(1, 128), lg.reshape(1, 128), lb.reshape(1, 128),
      wc2, bc2.reshape(1, NCLS))


def kernel(node_features, edge_index, batch, static_features, W_proj, b_proj,
           ln1_g, ln1_b, W1, att_src1, att_dst1, b1, W2, att_src2, att_dst2,
           b2, Wc1, bc1, lnc_g, lnc_b, Wc2, bc2):
    f32 = jnp.float32
    nfp = jnp.pad(node_features, ((0, NPAD - N), (0, 0)))
    loop = jnp.arange(N, dtype=jnp.int32)
    npadidx = jnp.full((EPAD - E - N,), NPAD - 1, jnp.int32)
    srcp = jnp.concatenate([edge_index[0].astype(jnp.int32), loop, npadidx])
    dstp = jnp.concatenate([edge_index[1].astype(jnp.int32), loop, npadidx])
    dstp2 = dstp.reshape(EPAD // EB, EB)
    batp = jnp.concatenate([batch.astype(jnp.int32),
                            jnp.full((NPAD - N,), B, jnp.int32)])
    bat3 = batp.reshape(NB, 1, 256)
    zh64 = jnp.zeros((ROWS_T, 64), f32)
    zh16 = jnp.zeros((ROWS_T, 16), f32)

    eye4 = jnp.eye(HEADS, dtype=f32)
    asm = (att_src1[:, :, None] * eye4[:, None, :]).reshape(HEADS * HID, HEADS)
    adm = (att_dst1[:, :, None] * eye4[:, None, :]).reshape(HEADS * HID, HEADS)

    h1c, as1, ad1 = _run_proj(nfp, W_proj, b_proj, ln1_g, ln1_b, W1, asm, adm)
    as1t = as1.T.reshape(HEADS, NPAD)
    ad1t = ad1.T.reshape(HEADS, NPAD)

    g1 = _run_gat1(as1t, ad1t, srcp, dstp2, zh64, zh16, h1c)
    den1f, out1 = g1[1], g1[2:2 + NCH1]
    den1r = den1f.reshape(2, NPAD, 16)
    den1s = jnp.stack([den1r[0, :, 0], den1r[0, :, 1],
                       den1r[1, :, 2], den1r[1, :, 3]])

    h2c, as2, ad2 = _run_mid(out1, den1s, b1, W2, att_src2, att_dst2)
    as2f = as2[:, 0]
    ad2f = ad2[:, 0]

    g2 = _run_gat2(as2f, ad2f, srcp, dstp2, zh64, zh16, h2c)
    den2f, out2 = g2[1], g2[2:2 + NCH2]
    den2s = den2f[:, 0].reshape(2, NPAD)

    return _run_head(out2, den2s, b2, bat3, static_features, Wc1, bc1,
                     lnc_g, lnc_b, Wc2, bc2)
